# baseline (speedup 1.0000x reference)
import functools

import jax
import jax.numpy as jnp
from jax import lax
from jax.experimental import pallas as pl
from jax.experimental.pallas import tpu as pltpu
from jax.experimental.pallas import tpu_sc as plsc

BATCH = 4096
SEQ = 200
EMBED = 32
R128 = 250000
NUM_WORKERS = 32
BPW = BATCH // NUM_WORKERS
NHALF = SEQ // 2

_mesh = plsc.VectorSubcoreMesh(core_axis_name="c", subcore_axis_name="s")


def _iota16(base):
    return lax.iota(jnp.int32, 16) + base


@functools.partial(
    pl.kernel,
    mesh=_mesh,
    compiler_params=pltpu.CompilerParams(
        use_tc_tiling_on_sc=True, needs_layout_passes=False),
    out_type=jax.ShapeDtypeStruct((SEQ, EMBED, BATCH), jnp.float32),
    scratch_types=[
        pltpu.VMEM((SEQ, 128), jnp.int32),
        pltpu.VMEM((128,), jnp.int32),
        pltpu.VMEM((128,), jnp.int32),
        pltpu.VMEM((128,), jnp.int32),
        pltpu.VMEM((128,), jnp.int32),
        pltpu.VMEM((128, 128), jnp.float32),
        pltpu.VMEM((128, 128), jnp.float32),
        pltpu.VMEM((4, 8, 128), jnp.float32),
        pltpu.VMEM((4, 8, 128), jnp.float32),
        pltpu.SemaphoreType.DMA,
        pltpu.SemaphoreType.DMA,
        pltpu.SemaphoreType.DMA,
        pltpu.SemaphoreType.DMA,
    ],
)
def _gather(idxt_hbm, tab_hbm, out_hbm,
            idxall, idxc0, idxc1, gidx0, gidx1, grows0, grows1,
            obuf0, obuf1, gsem0, gsem1, osem0, osem1):
    wid = lax.axis_index("s") * 2 + lax.axis_index("c")
    col = wid * BPW
    idxcs = (idxc0, idxc1)
    gidxs = (gidx0, gidx1)
    grows = (grows0, grows1)
    obufs = (obuf0, obuf1)
    gsems = (gsem0, gsem1)
    osems = (osem0, osem1)

    pltpu.sync_copy(idxt_hbm.at[:, pl.ds(col, BPW)], idxall)

    def prep_and_fire(s, b):
        for v in range(8):
            iv = idxall[s, pl.ds(16 * v, 16)]
            idxcs[b][pl.ds(16 * v, 16)] = iv
            gidxs[b][pl.ds(16 * v, 16)] = lax.shift_right_logical(iv, 2)
        pltpu.async_copy(tab_hbm.at[gidxs[b]], grows[b], gsems[b])

    def drain_gather(b):
        pltpu.make_async_copy(tab_hbm.at[gidxs[b]], grows[b], gsems[b]).wait()

    def compute(b):
        for v in range(8):
            iv = idxcs[b][pl.ds(16 * v, 16)]
            ov = lax.mul(lax.bitwise_and(iv, 3), 32)
            lv = _iota16(16 * v)
            for r_ in range(4):
                for cm in range(8):
                    c = 8 * r_ + cm
                    vals = plsc.load_gather(grows[b], [lv, ov + c])
                    obufs[b][r_, cm, pl.ds(16 * v, 16)] = vals

    def fire_out(s, b):
        for r_ in range(4):
            pltpu.async_copy(
                obufs[b].at[r_],
                out_hbm.at[s, pl.ds(8 * r_, 8), pl.ds(col, BPW)],
                osems[b],
            )

    def wait_out(b):
        for r_ in range(4):
            pltpu.make_async_copy(
                obufs[b].at[r_],
                out_hbm.at[0, pl.ds(8 * r_, 8), pl.ds(col, BPW)],
                osems[b],
            ).wait()

    prep_and_fire(0, 0)

    def t_body(t, carry):
        for b in (0, 1):
            s = 2 * t + b
            nb = 1 - b

            @pl.when(s + 1 < SEQ)
            def _():
                prep_and_fire(s + 1, nb)

            drain_gather(b)

            @pl.when(s >= 2)
            def _():
                wait_out(b)

            compute(b)
            fire_out(s, b)
        return carry

    lax.fori_loop(0, NHALF, t_body, 0)
    wait_out(0)
    wait_out(1)


def kernel(indices, table):
    idxt = indices.T
    tab128 = table.reshape(R128, 128)
    outp = _gather(idxt, tab128)
    return outp.transpose(2, 0, 1)


# speedup vs baseline: 1.1925x; 1.1925x over previous
import functools

import jax
import jax.numpy as jnp
from jax import lax
from jax.experimental import pallas as pl
from jax.experimental.pallas import tpu as pltpu
from jax.experimental.pallas import tpu_sc as plsc

BATCH = 4096
SEQ = 200
EMBED = 32
TOTAL = BATCH * SEQ
NUM_WORKERS = 32
PER_W = TOTAL // NUM_WORKERS
SUB = 1280
GROUP = 1280
G = GROUP // SUB
NGROUPS = PER_W // GROUP

_mesh = plsc.VectorSubcoreMesh(core_axis_name="c", subcore_axis_name="s")


@functools.partial(
    pl.kernel,
    mesh=_mesh,
    compiler_params=pltpu.CompilerParams(use_tc_tiling_on_sc=False),
    out_type=jax.ShapeDtypeStruct((TOTAL, EMBED), jnp.float32),
    scratch_types=[
        pltpu.VMEM((GROUP,), jnp.int32),
        pltpu.VMEM((GROUP,), jnp.int32),
        pltpu.VMEM((GROUP, EMBED), jnp.float32),
        pltpu.VMEM((GROUP, EMBED), jnp.float32),
        pltpu.SemaphoreType.DMA,
        pltpu.SemaphoreType.DMA,
        pltpu.SemaphoreType.DMA,
        pltpu.SemaphoreType.DMA,
    ],
)
def _gather(idx_hbm, table_hbm, out_hbm,
            idx0, idx1, rows0, rows1, gsem0, gsem1, osem0, osem1):
    wid = lax.axis_index("s") * 2 + lax.axis_index("c")
    base = wid * PER_W
    idx_bufs = (idx0, idx1)
    row_bufs = (rows0, rows1)
    gsems = (gsem0, gsem1)
    osems = (osem0, osem1)

    def load_idx(g, b):
        pltpu.sync_copy(idx_hbm.at[pl.ds(base + g * GROUP, GROUP)], idx_bufs[b])

    def fire(b):
        for j in range(G):
            pltpu.async_copy(
                table_hbm.at[idx_bufs[b].at[pl.ds(j * SUB, SUB)]],
                row_bufs[b].at[pl.ds(j * SUB, SUB)],
                gsems[b],
            )

    def drain(b):
        for j in range(G):
            pltpu.make_async_copy(
                out_hbm.at[pl.ds(0, SUB)],
                row_bufs[b].at[pl.ds(j * SUB, SUB)],
                gsems[b],
            ).wait()

    def fire_out(g, b):
        pltpu.async_copy(
            row_bufs[b], out_hbm.at[pl.ds(base + g * GROUP, GROUP)], osems[b])

    def wait_out(b):
        pltpu.make_async_copy(
            row_bufs[b], out_hbm.at[pl.ds(base, GROUP)], osems[b]).wait()

    load_idx(0, 0)
    fire(0)

    def t_body(t, carry):
        for b in (0, 1):
            g = 2 * t + b
            nb = 1 - b
            not_last = g + 1 < NGROUPS

            @pl.when(not_last)
            def _():
                load_idx(g + 1, nb)

            @pl.when(g >= 1)
            def _():
                wait_out(nb)

            @pl.when(not_last)
            def _():
                fire(nb)

            drain(b)
            fire_out(g, b)
        return carry

    lax.fori_loop(0, NGROUPS // 2, t_body, 0)
    wait_out(1)


def kernel(indices, table):
    idx_flat = indices.reshape(TOTAL)
    out = _gather(idx_flat, table)
    return out.reshape(BATCH, SEQ, EMBED)
